# baseline (speedup 1.0000x reference)
import jax
import jax.numpy as jnp
from jax.experimental import pallas as pl
from jax.experimental.pallas import tpu as pltpu

_T = 0.05
_BLK = 3072


def _mm_kernel(x_ref, m_ref, o_ref):
    acc = jax.lax.dot_general(
        x_ref[...], m_ref[...],
        dimension_numbers=(((1,), (1,)), ((), ())),
        preferred_element_type=jnp.float32)
    o_ref[...] = acc / _T


def kernel(x, memory):
    b, k = x.shape
    n = memory.shape[0]
    return pl.pallas_call(
        _mm_kernel,
        grid=(pl.cdiv(n, _BLK),),
        in_specs=[
            pl.BlockSpec((b, k), lambda i: (0, 0)),
            pl.BlockSpec((_BLK, k), lambda i: (i, 0)),
        ],
        out_specs=pl.BlockSpec((b, _BLK), lambda i: (0, i)),
        out_shape=jax.ShapeDtypeStruct((b, n), jnp.float32),
        compiler_params=pltpu.CompilerParams(
            dimension_semantics=("arbitrary",),
            vmem_limit_bytes=100 * 1024 * 1024),
    )(x, memory)


# speedup vs baseline: 1.0118x; 1.0118x over previous
import functools

import jax
import jax.numpy as jnp
from jax import lax
from jax.experimental import pallas as pl
from jax.experimental.pallas import tpu as pltpu
from jax.experimental.pallas import tpu_sc as plsc

_T = 0.05
_BLK = 2048


def _mm_kernel(x_ref, m_ref, o_ref):
    acc = jax.lax.dot_general(
        x_ref[...], m_ref[...],
        dimension_numbers=(((1,), (1,)), ((), ())),
        precision=jax.lax.Precision.DEFAULT,
        preferred_element_type=jnp.float32)
    o_ref[...] = acc / _T


_NC, _NS = 2, 16
_NW = _NC * _NS
_SC_CHUNK = 32
_SC_ROWS = 98304


def _sc_stream_probe(memory):
    rows_per_w = _SC_ROWS // _NW
    n_iters = rows_per_w // _SC_CHUNK
    mesh = plsc.VectorSubcoreMesh(core_axis_name="c", subcore_axis_name="s")

    @functools.partial(
        pl.kernel,
        out_type=jax.ShapeDtypeStruct((_NW, 2048), jnp.float32),
        mesh=mesh,
        scratch_types=[pltpu.VMEM((_SC_CHUNK, 2048), jnp.float32)],
    )
    def k(mem_hbm, out_hbm, buf):
        wid = lax.axis_index("s") * _NC + lax.axis_index("c")
        base = wid * rows_per_w

        def body(i, c):
            pltpu.sync_copy(mem_hbm.at[pl.ds(base + i * _SC_CHUNK, _SC_CHUNK)], buf)
            return c

        lax.fori_loop(0, n_iters, body, 0)
        pltpu.sync_copy(buf.at[0], out_hbm.at[wid])

    return k(memory)


def kernel(x, memory):
    return _tc_kernel(x, memory)


def _tc_kernel(x, memory):
    b, k = x.shape
    n = memory.shape[0]
    return pl.pallas_call(
        _mm_kernel,
        grid=(pl.cdiv(n, _BLK),),
        in_specs=[
            pl.BlockSpec((b, k), lambda i: (0, 0)),
            pl.BlockSpec((_BLK, k), lambda i: (i, 0)),
        ],
        out_specs=pl.BlockSpec((b, _BLK), lambda i: (0, i)),
        out_shape=jax.ShapeDtypeStruct((b, n), jnp.float32),
        compiler_params=pltpu.CompilerParams(
            dimension_semantics=("arbitrary",),
            vmem_limit_bytes=100 * 1024 * 1024),
    )(x, memory)
